# baseline (speedup 1.0000x reference)
import functools

import jax
import jax.numpy as jnp
from jax import lax
from jax.experimental import pallas as pl
from jax.experimental.pallas import tpu as pltpu
from jax.experimental.pallas import tpu_sc as plsc

N, C, H, W = 4, 96, 384, 384
YA, XA = 64, 64
HBLK = 32
HB = H // HBLK
M = H * W
WCH = 128
NLANE = 16
_JSETS = {0: (0, 1), 1: (0, 1, 2), 2: (1, 2)}


def _pre_body(info_ref, x_ref, w_ref, b_ref, o_ref, stats_ref,
              stash, sum_s, sq_s, aT, b2T):
    t = pl.program_id(0)
    hb = pl.program_id(1)
    is_read = t < 3
    is_apply = t >= 1

    @pl.when(is_apply)
    def _apply_phase():
        xf = stash[:, pl.ds(hb * HBLK, HBLK), :].astype(jnp.float32)
        a = jnp.where(t == 1, aT[:, 0:1, :],
                      jnp.where(t == 2, aT[:, 1:2, :], aT[:, 2:3, :]))
        b2 = jnp.where(t == 1, b2T[:, 0:1, :],
                       jnp.where(t == 2, b2T[:, 1:2, :], b2T[:, 2:3, :]))
        o_ref[0] = xf * a + b2

    @pl.when(is_read)
    def _read_phase():
        xb = x_ref[0]
        stash[:, pl.ds(hb * HBLK, HBLK), :] = xb.astype(jnp.bfloat16)
        s = jnp.sum(xb, axis=(1, 2))[None, :]
        q = jnp.sum(xb * xb, axis=(1, 2))[None, :]

        @pl.when(hb == 0)
        def _init():
            sum_s[pl.ds(t, 1), :] = s
            sq_s[pl.ds(t, 1), :] = q

        @pl.when(hb != 0)
        def _acc():
            sum_s[pl.ds(t, 1), :] = sum_s[pl.ds(t, 1), :] + s
            sq_s[pl.ds(t, 1), :] = sq_s[pl.ds(t, 1), :] + q

        @pl.when(hb == HB - 1)
        def _finalize():
            sm = sum_s[pl.ds(t, 1), :]
            mean_i = sm * (1.0 / M)
            var_i = (sq_s[pl.ds(t, 1), :] - sm * mean_i) * (1.0 / (M - 1))
            std_i = jnp.sqrt(var_i)
            stats_ref[pl.ds(t, 1), :] = mean_i
            stats_ref[pl.ds(3 + t, 1), :] = std_i
            a_i = w_ref[:] / std_i
            b2_i = b_ref[:] - mean_i * a_i
            ac = a_i.reshape(C, 1, 1)
            bc = b2_i.reshape(C, 1, 1)
            for tt in range(N - 1):
                @pl.when(t == tt)
                def _store():
                    aT[:, tt:tt + 1, :] = ac
                    b2T[:, tt:tt + 1, :] = bc


def _x_map(t, hb):
    tile = jnp.minimum(t + 1, 3)
    hbi = jnp.where(t == 3, HB - 1, hb)
    return (tile, 0, hbi, 0)


def _o_map(t, hb):
    tile = jnp.maximum(t, 1)
    hbo = jnp.where(t == 0, 0, hb)
    return (tile, 0, hbo, 0)


def _sc_window_body(info_hbm, tm_hbm, ts_hbm, stats_hbm, outm_hbm, outs_hbm,
                    info_v, stats_v, idx_v, mrows_v, srows_v,
                    outm_v, outs_v, sem):
    cid = lax.axis_index("c")
    sid = lax.axis_index("s")

    def bcast(vec, j):
        return vec.at[jnp.full((NLANE,), j, jnp.int32)].get(
            mode="promise_in_bounds")

    @pl.when((cid == 0) & (sid == 0))
    def _():
        pltpu.sync_copy(info_hbm, info_v)
        pltpu.sync_copy(stats_hbm, stats_v)
        infov = info_v[...]
        yc = bcast(infov, 0)
        xc = bcast(infov, 1)
        lane = lax.iota(jnp.int32, NLANE)
        dyv = lax.shift_right_logical(lane * 11, 5)
        dxv = lane - dyv * 3
        uyv = jnp.clip(yc - 2 + dyv, 0, YA - 1)
        uxv = jnp.clip(xc - 2 + dxv, 0, XA - 1)
        idx_v[...] = uyv * XA + uxv
        pltpu.async_copy(tm_hbm.at[idx_v], mrows_v, sem).wait()
        pltpu.async_copy(ts_hbm.at[idx_v], srows_v, sem).wait()
        hitv = []
        for i in range(N - 1):
            pyi = bcast(infov, 2 + i)
            pxi = bcast(infov, 5 + i)
            hitv.append((pyi != -1) & (pyi == uyv) & (pxi == uxv))
        for c6 in range(C // NLANE):
            sl = pl.ds(c6 * NLANE, NLANE)
            m4 = mrows_v[4, sl]
            s4 = srows_v[4, sl]
            for i in range(N - 1):
                h4 = bcast(hitv[i].astype(jnp.int32), 4) != 0
                m4 = jnp.where(h4, stats_v[i, sl], m4)
                s4 = jnp.where(h4, stats_v[3 + i, sl], s4)
            for k in range(9):
                m = mrows_v[k, sl]
                s = srows_v[k, sl]
                for i in range(N - 1):
                    hk = bcast(hitv[i].astype(jnp.int32), k) != 0
                    m = jnp.where(hk, stats_v[i, sl], m)
                    s = jnp.where(hk, stats_v[3 + i, sl], s)
                m = jnp.where(m == 0.0, m4, m)
                s = jnp.where(s == 0.0, s4, s)
                pos = lane * 9 + (c6 * NLANE * 9 + k)
                plsc.store_scatter(outm_v, [pos], m)
                plsc.store_scatter(outs_v, [pos], s)
        pltpu.sync_copy(outm_v, outm_hbm)
        pltpu.sync_copy(outs_v, outs_hbm)


def _real_body(x_ref, prev_ref, wm_ref, ws_ref, w_ref, b_ref, o_ref):
    hb = pl.program_id(0)
    wv = w_ref[:].reshape(C, 1, 1)
    bv = b_ref[:].reshape(C, 1, 1)
    ry = jax.lax.broadcasted_iota(jnp.int32, (1, HBLK, 1), 1).astype(jnp.float32)
    sy = 0.5 + (hb * HBLK + ry + 0.5) * (1.0 / H)
    gys = [jnp.maximum(0.0, 1.0 - jnp.abs(sy - i)) for i in range(3)]
    cms = {}
    css = {}
    for j in range(3):
        cm = jnp.zeros((C, HBLK, 1), jnp.float32)
        cs = jnp.zeros((C, HBLK, 1), jnp.float32)
        for i in range(3):
            k = 3 * i + j
            cm = cm + wm_ref[:, k:k + 1, :] * gys[i]
            cs = cs + ws_ref[:, k:k + 1, :] * gys[i]
        cms[j] = cm
        css[j] = cs
    for ci, wc in enumerate(range(0, W, WCH)):
        cx = jax.lax.broadcasted_iota(jnp.int32, (1, 1, WCH), 2).astype(jnp.float32)
        sx = 0.5 + (wc + cx + 0.5) * (1.0 / W)
        mmap = jnp.zeros((C, HBLK, WCH), jnp.float32)
        smap = jnp.zeros((C, HBLK, WCH), jnp.float32)
        for j in _JSETS[ci]:
            gx = jnp.maximum(0.0, 1.0 - jnp.abs(sx - j))
            mmap = mmap + cms[j] * gx
            smap = smap + css[j] * gx
        xc_blk = x_ref[0, :, :, pl.ds(wc, WCH)]
        o_ref[0, :, :, pl.ds(wc, WCH)] = (xc_blk - mmap) * (1.0 / smap) * wv + bv


def kernel(x, weight, bias, mean_table, std_table, pre_y_anchor, pre_x_anchor,
           y_anchor, x_anchor, padding):
    f32 = jnp.float32
    info = jnp.concatenate([
        jnp.stack([y_anchor + padding, x_anchor + padding]).astype(jnp.int32),
        pre_y_anchor.astype(jnp.int32),
        pre_x_anchor.astype(jnp.int32),
    ])
    tm = jnp.pad(mean_table.reshape(YA * XA, C), ((0, 0), (0, 128 - C)))
    ts = jnp.pad(std_table.reshape(YA * XA, C), ((0, 0), (0, 128 - C)))
    w2 = weight.reshape(1, C)
    b2in = bias.reshape(1, C)

    out1, stats = pl.pallas_call(
        _pre_body,
        grid=(N, HB),
        in_specs=[pl.BlockSpec(memory_space=pltpu.SMEM),
                  pl.BlockSpec((1, C, HBLK, W), _x_map),
                  pl.BlockSpec((1, C), lambda t, hb: (0, 0)),
                  pl.BlockSpec((1, C), lambda t, hb: (0, 0))],
        out_specs=[pl.BlockSpec((1, C, HBLK, W), _o_map),
                   pl.BlockSpec((2 * (N - 1), C), lambda t, hb: (0, 0))],
        out_shape=[jax.ShapeDtypeStruct((N, C, H, W), f32),
                   jax.ShapeDtypeStruct((2 * (N - 1), C), f32)],
        scratch_shapes=[pltpu.VMEM((C, H, W), jnp.bfloat16),
                        pltpu.VMEM((N - 1, C), f32),
                        pltpu.VMEM((N - 1, C), f32),
                        pltpu.VMEM((C, N - 1, 1), f32),
                        pltpu.VMEM((C, N - 1, 1), f32)],
        compiler_params=pltpu.CompilerParams(vmem_limit_bytes=67108864),
    )(info, x, w2, b2in)

    sc_window = functools.partial(
        pl.kernel,
        out_type=[jax.ShapeDtypeStruct((C * 9,), f32),
                  jax.ShapeDtypeStruct((C * 9,), f32)],
        mesh=plsc.VectorSubcoreMesh(core_axis_name="c", subcore_axis_name="s"),
        compiler_params=pltpu.CompilerParams(needs_layout_passes=False),
        scratch_types=[pltpu.VMEM((NLANE,), jnp.int32),
                       pltpu.VMEM((2 * (N - 1), C), f32),
                       pltpu.VMEM((NLANE,), jnp.int32),
                       pltpu.VMEM((NLANE, 128), f32),
                       pltpu.VMEM((NLANE, 128), f32),
                       pltpu.VMEM((C * 9,), f32),
                       pltpu.VMEM((C * 9,), f32),
                       pltpu.SemaphoreType.DMA],
    )(_sc_window_body)
    info16 = jnp.concatenate([info, jnp.zeros((NLANE - 8,), jnp.int32)])
    wmflat, wsflat = sc_window(info16, tm, ts, stats)
    wmT = wmflat.reshape(C, 9, 1)
    wsT = wsflat.reshape(C, 9, 1)

    out = pl.pallas_call(
        _real_body,
        grid=(HB,),
        in_specs=[pl.BlockSpec((1, C, HBLK, W), lambda hb: (0, 0, hb, 0)),
                  pl.BlockSpec(memory_space=pl.ANY),
                  pl.BlockSpec((C, 9, 1), lambda hb: (0, 0, 0)),
                  pl.BlockSpec((C, 9, 1), lambda hb: (0, 0, 0)),
                  pl.BlockSpec((1, C), lambda hb: (0, 0)),
                  pl.BlockSpec((1, C), lambda hb: (0, 0))],
        out_specs=pl.BlockSpec((1, C, HBLK, W), lambda hb: (0, 0, hb, 0)),
        out_shape=jax.ShapeDtypeStruct((N, C, H, W), f32),
        input_output_aliases={1: 0},
        compiler_params=pltpu.CompilerParams(vmem_limit_bytes=67108864),
    )(x, out1, wmT, wsT, w2, b2in)
    return out


# speedup vs baseline: 1.1722x; 1.1722x over previous
import jax
import jax.numpy as jnp
from jax.experimental import pallas as pl
from jax.experimental.pallas import tpu as pltpu

N, C, H, W = 4, 96, 384, 384
YA, XA = 64, 64
HBLK = 32
HB = H // HBLK
M = H * W
WCH = 128
_JSETS = {0: (0, 1), 1: (0, 1, 2), 2: (1, 2)}


def _fused_body(info_ref, x_ref, tm_ref, ts_ref, w_ref, b_ref, o_ref,
                stash, sum_s, sq_s, mean_s, std_s, aT, b2T, wmT, wsT,
                trow_s, dma_sem):
    t = pl.program_id(0)
    hb = pl.program_id(1)
    is_read = t < 3
    is_apply = (t >= 1) & (t <= 3)

    @pl.when(is_apply)
    def _apply_phase():
        xf = stash[:, pl.ds(hb * HBLK, HBLK), :].astype(jnp.float32)
        a = jnp.where(t == 1, aT[:, 0:1, :],
                      jnp.where(t == 2, aT[:, 1:2, :], aT[:, 2:3, :]))
        b2 = jnp.where(t == 1, b2T[:, 0:1, :],
                       jnp.where(t == 2, b2T[:, 1:2, :], b2T[:, 2:3, :]))
        o_ref[0] = xf * a + b2

    @pl.when(is_read)
    def _read_phase():
        xb = x_ref[0]
        stash[:, pl.ds(hb * HBLK, HBLK), :] = xb.astype(jnp.bfloat16)
        s = jnp.sum(xb, axis=(1, 2))[None, :]
        q = jnp.sum(xb * xb, axis=(1, 2))[None, :]

        @pl.when(hb == 0)
        def _init():
            sum_s[pl.ds(t, 1), :] = s
            sq_s[pl.ds(t, 1), :] = q

        @pl.when(hb != 0)
        def _acc():
            sum_s[pl.ds(t, 1), :] = sum_s[pl.ds(t, 1), :] + s
            sq_s[pl.ds(t, 1), :] = sq_s[pl.ds(t, 1), :] + q

        @pl.when(hb == HB - 1)
        def _finalize():
            sm = sum_s[pl.ds(t, 1), :]
            mean_i = sm * (1.0 / M)
            var_i = (sq_s[pl.ds(t, 1), :] - sm * mean_i) * (1.0 / (M - 1))
            std_i = jnp.sqrt(var_i)
            mean_s[pl.ds(t, 1), :] = mean_i
            std_s[pl.ds(t, 1), :] = std_i
            a_i = w_ref[:] / std_i
            b2_i = b_ref[:] - mean_i * a_i
            ac = a_i.reshape(C, 1, 1)
            bc = b2_i.reshape(C, 1, 1)
            for tt in range(N - 1):
                @pl.when(t == tt)
                def _store():
                    aT[:, tt:tt + 1, :] = ac
                    b2T[:, tt:tt + 1, :] = bc

        @pl.when((t == 2) & (hb == HB - 1))
        def _window():
            mean = mean_s[:]
            std = std_s[:]
            yc = info_ref[0]
            xc = info_ref[1]
            uys = []
            uxs = []
            for k in range(9):
                dy, dx = k // 3, k % 3
                uys.append(jnp.clip(yc - 2 + dy, 0, YA - 1))
                uxs.append(jnp.clip(xc - 2 + dx, 0, XA - 1))
            for k in range(9):
                r = uys[k] * XA + uxs[k]
                pltpu.make_async_copy(tm_ref.at[pl.ds(r, 1), :],
                                      trow_s.at[pl.ds(k, 1), :],
                                      dma_sem).start()
                pltpu.make_async_copy(ts_ref.at[pl.ds(r, 1), :],
                                      trow_s.at[pl.ds(9 + k, 1), :],
                                      dma_sem).start()
            for k in range(9):
                r = uys[k] * XA + uxs[k]
                pltpu.make_async_copy(tm_ref.at[pl.ds(r, 1), :],
                                      trow_s.at[pl.ds(k, 1), :],
                                      dma_sem).wait()
                pltpu.make_async_copy(ts_ref.at[pl.ds(r, 1), :],
                                      trow_s.at[pl.ds(9 + k, 1), :],
                                      dma_sem).wait()
            mrows = []
            srows = []
            for k in range(9):
                mrow = trow_s[pl.ds(k, 1), :]
                srow = trow_s[pl.ds(9 + k, 1), :]
                for i in range(N - 1):
                    hit = ((info_ref[2 + i] != -1)
                           & (info_ref[2 + i] == uys[k])
                           & (info_ref[5 + i] == uxs[k]))
                    mrow = jnp.where(hit, mean[i][None, :], mrow)
                    srow = jnp.where(hit, std[i][None, :], srow)
                mrows.append(mrow)
                srows.append(srow)
            mwin = jnp.concatenate(mrows, axis=0)
            swin = jnp.concatenate(srows, axis=0)
            mwin = jnp.where(mwin == 0.0, mwin[4:5], mwin)
            swin = jnp.where(swin == 0.0, swin[4:5], swin)
            for k in range(9):
                wmT[:, k:k + 1, :] = mwin[k:k + 1].reshape(C, 1, 1)
                wsT[:, k:k + 1, :] = swin[k:k + 1].reshape(C, 1, 1)

    @pl.when(t == 4)
    def _real_phase():
        wv = w_ref[:].reshape(C, 1, 1)
        bv = b_ref[:].reshape(C, 1, 1)
        ry = jax.lax.broadcasted_iota(jnp.int32, (1, HBLK, 1), 1).astype(jnp.float32)
        sy = 0.5 + (hb * HBLK + ry + 0.5) * (1.0 / H)
        gys = [jnp.maximum(0.0, 1.0 - jnp.abs(sy - i)) for i in range(3)]
        cms = {}
        css = {}
        for j in range(3):
            cm = jnp.zeros((C, HBLK, 1), jnp.float32)
            cs = jnp.zeros((C, HBLK, 1), jnp.float32)
            for i in range(3):
                k = 3 * i + j
                cm = cm + wmT[:, k:k + 1, :] * gys[i]
                cs = cs + wsT[:, k:k + 1, :] * gys[i]
            cms[j] = cm
            css[j] = cs
        for ci, wc in enumerate(range(0, W, WCH)):
            cx = jax.lax.broadcasted_iota(jnp.int32, (1, 1, WCH), 2).astype(jnp.float32)
            sx = 0.5 + (wc + cx + 0.5) * (1.0 / W)
            mmap = jnp.zeros((C, HBLK, WCH), jnp.float32)
            smap = jnp.zeros((C, HBLK, WCH), jnp.float32)
            for j in _JSETS[ci]:
                gx = jnp.maximum(0.0, 1.0 - jnp.abs(sx - j))
                mmap = mmap + cms[j] * gx
                smap = smap + css[j] * gx
            xc_blk = x_ref[0, :, :, pl.ds(wc, WCH)]
            o_ref[0, :, :, pl.ds(wc, WCH)] = (xc_blk - mmap) * (1.0 / smap) * wv + bv


def _x_map(t, hb):
    tile = jnp.where(t == 4, 0, jnp.minimum(t + 1, 3))
    hbi = jnp.where(t == 3, HB - 1, hb)
    return (tile, 0, hbi, 0)


def _o_map(t, hb):
    tile = jnp.where(t == 4, 0, jnp.maximum(t, 1))
    hbo = jnp.where(t == 0, 0, hb)
    return (tile, 0, hbo, 0)


def kernel(x, weight, bias, mean_table, std_table, pre_y_anchor, pre_x_anchor,
           y_anchor, x_anchor, padding):
    f32 = jnp.float32
    info = jnp.concatenate([
        jnp.stack([y_anchor + padding, x_anchor + padding]).astype(jnp.int32),
        pre_y_anchor.astype(jnp.int32),
        pre_x_anchor.astype(jnp.int32),
    ])
    tm = mean_table.reshape(YA * XA, C)
    ts = std_table.reshape(YA * XA, C)
    w2 = weight.reshape(1, C)
    b2in = bias.reshape(1, C)

    out = pl.pallas_call(
        _fused_body,
        grid=(N + 1, HB),
        in_specs=[pl.BlockSpec(memory_space=pltpu.SMEM),
                  pl.BlockSpec((1, C, HBLK, W), _x_map),
                  pl.BlockSpec(memory_space=pl.ANY),
                  pl.BlockSpec(memory_space=pl.ANY),
                  pl.BlockSpec((1, C), lambda t, hb: (0, 0)),
                  pl.BlockSpec((1, C), lambda t, hb: (0, 0))],
        out_specs=pl.BlockSpec((1, C, HBLK, W), _o_map),
        out_shape=jax.ShapeDtypeStruct((N, C, H, W), f32),
        scratch_shapes=[pltpu.VMEM((C, H, W), jnp.bfloat16),
                        pltpu.VMEM((N - 1, C), f32),
                        pltpu.VMEM((N - 1, C), f32),
                        pltpu.VMEM((N - 1, C), f32),
                        pltpu.VMEM((N - 1, C), f32),
                        pltpu.VMEM((C, N - 1, 1), f32),
                        pltpu.VMEM((C, N - 1, 1), f32),
                        pltpu.VMEM((C, 9, 1), f32),
                        pltpu.VMEM((C, 9, 1), f32),
                        pltpu.VMEM((18, C), f32),
                        pltpu.SemaphoreType.DMA],
        compiler_params=pltpu.CompilerParams(vmem_limit_bytes=67108864),
    )(info, x, tm, ts, w2, b2in)
    return out
